# Initial kernel scaffold; baseline (speedup 1.0000x reference)
import functools
import math

import jax
import jax.numpy as jnp
from jax import lax
from jax.experimental import pallas as pl
from jax.experimental.pallas import tpu as pltpu
from jax.experimental.pallas import tpu_sc as plsc

_D = 64
_S2_BITS = 20
_S2_MASK = (1 << _S2_BITS) - 1
_NW = 32
_C = 128


def _proj_kernel(emb_ref, w_ref, out_ref):
    out_ref[...] = jnp.dot(emb_ref[...], w_ref[...],
                           preferred_element_type=jnp.float32)


def _proj_bias_kernel(emb_ref, w_ref, b_ref, out_ref):
    out_ref[...] = (jnp.dot(emb_ref[...], w_ref[...],
                            preferred_element_type=jnp.float32) + b_ref[...])


@functools.lru_cache(maxsize=None)
def _make_sc_gather(n_tokens: int):
    assert n_tokens % (_NW * _C) == 0
    per_w = n_tokens // _NW
    n_chunks = per_w // _C
    mesh = plsc.VectorSubcoreMesh(core_axis_name="c", subcore_axis_name="s")

    @functools.partial(
        pl.kernel,
        out_type=jax.ShapeDtypeStruct((n_tokens, _D), jnp.float32),
        mesh=mesh,
        scratch_types=[
            pltpu.VMEM((_C,), jnp.int32),
            pltpu.VMEM((_C,), jnp.int32),
            pltpu.VMEM((_C,), jnp.int32),
            pltpu.VMEM((_C, _D), jnp.float32),
            pltpu.VMEM((_C, _D), jnp.float32),
            pltpu.SemaphoreType.DMA,
            pltpu.SemaphoreType.DMA,
        ],
    )
    def sc_gather(tid_hbm, p1_hbm, p2_hbm, out_hbm,
                  tid_v, s1_v, s2_v, buf1, buf2, sem1, sem2):
        wid = lax.axis_index("s") * 2 + lax.axis_index("c")
        wbase = wid * per_w

        def chunk_body(ci, carry):
            base = wbase + ci * _C
            pltpu.sync_copy(tid_hbm.at[pl.ds(base, _C)], tid_v)

            def idx_body(i, c):
                sl = pl.ds(i * 16, 16)
                t = tid_v[sl]
                s1_v[sl] = lax.shift_right_logical(t, _S2_BITS)
                s2_v[sl] = lax.bitwise_and(t, _S2_MASK)
                return c

            lax.fori_loop(0, _C // 16, idx_body, 0)
            cp1 = pltpu.async_copy(p1_hbm.at[s1_v], buf1, sem1)
            cp2 = pltpu.async_copy(p2_hbm.at[s2_v], buf2, sem2)
            cp1.wait()
            cp2.wait()

            def add_body(i, c):
                for k in range(_D // 16):
                    sl = pl.ds(k * 16, 16)
                    buf2[i, sl] = buf2[i, sl] + buf1[i, sl]
                return c

            lax.fori_loop(0, _C, add_body, 0)
            pltpu.sync_copy(buf2, out_hbm.at[pl.ds(base, _C)])
            return carry

        lax.fori_loop(0, n_chunks, chunk_body, 0)

    return sc_gather


def kernel(token_ids, emb_s1, emb_s2, W, b):
    B, L = token_ids.shape
    n = B * L
    scale = math.sqrt(_D)
    w1 = W[:_D] * scale
    w2 = W[_D:] * scale

    p1 = pl.pallas_call(
        _proj_bias_kernel,
        out_shape=jax.ShapeDtypeStruct((emb_s1.shape[0], _D), jnp.float32),
    )(emb_s1, w1, b.reshape(1, _D))

    blk = 8192
    n2 = emb_s2.shape[0]
    p2 = pl.pallas_call(
        _proj_kernel,
        grid=(n2 // blk,),
        in_specs=[
            pl.BlockSpec((blk, _D), lambda i: (i, 0)),
            pl.BlockSpec((_D, _D), lambda i: (0, 0)),
        ],
        out_specs=pl.BlockSpec((blk, _D), lambda i: (i, 0)),
        out_shape=jax.ShapeDtypeStruct((n2, _D), jnp.float32),
    )(emb_s2, w2)

    out = _make_sc_gather(n)(token_ids.reshape(n), p1, p2)
    return out.reshape(B, L, _D)


# speedup vs baseline: 1.6799x; 1.6799x over previous
import functools
import math

import jax
import jax.numpy as jnp
from jax import lax
from jax.experimental import pallas as pl
from jax.experimental.pallas import tpu as pltpu
from jax.experimental.pallas import tpu_sc as plsc

_D = 64
_S2_BITS = 20
_S2_MASK = (1 << _S2_BITS) - 1
_NW = 32
_C = 128


def _proj_kernel(emb_ref, w_ref, out_ref):
    out_ref[...] = jnp.dot(emb_ref[...], w_ref[...],
                           preferred_element_type=jnp.float32)


def _proj_bias_kernel(emb_ref, w_ref, b_ref, out_ref):
    out_ref[...] = (jnp.dot(emb_ref[...], w_ref[...],
                            preferred_element_type=jnp.float32) + b_ref[...])


@functools.lru_cache(maxsize=None)
def _make_sc_gather(n_tokens: int):
    assert n_tokens % (_NW * _C) == 0
    per_w = n_tokens // _NW
    n_chunks = per_w // _C
    mesh = plsc.VectorSubcoreMesh(core_axis_name="c", subcore_axis_name="s")

    @functools.partial(
        pl.kernel,
        out_type=jax.ShapeDtypeStruct((n_tokens, _D), jnp.float32),
        mesh=mesh,
        compiler_params=pltpu.CompilerParams(use_tc_tiling_on_sc=False),
        scratch_types=[
            pltpu.VMEM((_C,), jnp.int32),
            pltpu.VMEM((_C,), jnp.int32),
            pltpu.VMEM((_C,), jnp.int32),
            pltpu.VMEM((_C, _D), jnp.float32),
            pltpu.VMEM((_C, _D), jnp.float32),
            pltpu.SemaphoreType.DMA,
            pltpu.SemaphoreType.DMA,
        ],
    )
    def sc_gather(tid_hbm, p1_hbm, p2_hbm, out_hbm,
                  tid_v, s1_v, s2_v, buf1, buf2, sem1, sem2):
        wid = lax.axis_index("s") * 2 + lax.axis_index("c")
        wbase = wid * per_w

        def chunk_body(ci, carry):
            base = wbase + ci * _C
            pltpu.sync_copy(tid_hbm.at[pl.ds(base, _C)], tid_v)

            def idx_body(i, c):
                sl = pl.ds(i * 16, 16)
                t = tid_v[sl]
                s1_v[sl] = lax.shift_right_logical(t, _S2_BITS)
                s2_v[sl] = lax.bitwise_and(t, _S2_MASK)
                return c

            lax.fori_loop(0, _C // 16, idx_body, 0)
            cp1 = pltpu.async_copy(p1_hbm.at[s1_v], buf1, sem1)
            cp2 = pltpu.async_copy(p2_hbm.at[s2_v], buf2, sem2)
            cp1.wait()
            cp2.wait()

            def add_body(i, c):
                for k in range(_D // 16):
                    sl = pl.ds(k * 16, 16)
                    buf2[i, sl] = buf2[i, sl] + buf1[i, sl]
                return c

            lax.fori_loop(0, _C, add_body, 0)
            pltpu.sync_copy(buf2, out_hbm.at[pl.ds(base, _C)])
            return carry

        lax.fori_loop(0, n_chunks, chunk_body, 0)

    return sc_gather


def kernel(token_ids, emb_s1, emb_s2, W, b):
    B, L = token_ids.shape
    n = B * L
    scale = math.sqrt(_D)
    w1 = W[:_D] * scale
    w2 = W[_D:] * scale

    p1 = pl.pallas_call(
        _proj_bias_kernel,
        out_shape=jax.ShapeDtypeStruct((emb_s1.shape[0], _D), jnp.float32),
    )(emb_s1, w1, b.reshape(1, _D))

    blk = 8192
    n2 = emb_s2.shape[0]
    p2 = pl.pallas_call(
        _proj_kernel,
        grid=(n2 // blk,),
        in_specs=[
            pl.BlockSpec((blk, _D), lambda i: (i, 0)),
            pl.BlockSpec((_D, _D), lambda i: (0, 0)),
        ],
        out_specs=pl.BlockSpec((blk, _D), lambda i: (i, 0)),
        out_shape=jax.ShapeDtypeStruct((n2, _D), jnp.float32),
    )(emb_s2, w2)

    out = _make_sc_gather(n)(token_ids.reshape(n), p1, p2)
    return out.reshape(B, L, _D)


# speedup vs baseline: 2.8720x; 1.7096x over previous
import functools
import math

import jax
import jax.numpy as jnp
from jax import lax
from jax.experimental import pallas as pl
from jax.experimental.pallas import tpu as pltpu
from jax.experimental.pallas import tpu_sc as plsc

_D = 64
_S2_BITS = 20
_S2_MASK = (1 << _S2_BITS) - 1
_NW = 32
_C = 200
_SPLIT = 104


def _proj_t_kernel(embt_ref, w_ref, out_ref):
    out_ref[...] = lax.dot_general(embt_ref[...], w_ref[...],
                                   (((0,), (0,)), ((), ())),
                                   preferred_element_type=jnp.float32)


def _proj_t_bias_kernel(embt_ref, w_ref, b_ref, out_ref):
    out_ref[...] = lax.dot_general(embt_ref[...], w_ref[...],
                                   (((0,), (0,)), ((), ())),
                                   preferred_element_type=jnp.float32) + b_ref[...]


@functools.lru_cache(maxsize=None)
def _make_sc_gather(batch: int, seq: int):
    assert seq == _C and batch % _NW == 0
    seq_per_w = batch // _NW
    mesh = plsc.VectorSubcoreMesh(core_axis_name="c", subcore_axis_name="s")

    @functools.partial(
        pl.kernel,
        out_type=jax.ShapeDtypeStruct((batch, seq, _D), jnp.float32),
        mesh=mesh,
        compiler_params=pltpu.CompilerParams(use_tc_tiling_on_sc=False),
        scratch_types=[
            pltpu.VMEM((_C,), jnp.int32),
            pltpu.VMEM((_C,), jnp.int32),
            pltpu.VMEM((_C,), jnp.int32),
            pltpu.VMEM((_C, _D), jnp.float32),
            pltpu.VMEM((_C, _D), jnp.float32),
            pltpu.SemaphoreType.DMA,
            pltpu.SemaphoreType.DMA,
        ],
    )
    def sc_gather(tid_hbm, p1_hbm, p2_hbm, out_hbm,
                  tid_v, s1_v, s2_v, buf1, buf2, sem1, sem2):
        wid = lax.axis_index("s") * 2 + lax.axis_index("c")
        sbase = wid * seq_per_w

        def chunk_body(ci, carry):
            b = sbase + ci
            pltpu.sync_copy(tid_hbm.at[pl.ds(b * seq, _C)], tid_v)

            starts = list(range(0, _C - 15, 16))
            if starts[-1] != _C - 16:
                starts.append(_C - 16)
            for start in starts:
                sl = pl.ds(start, 16)
                t = tid_v[sl]
                s1_v[sl] = lax.shift_right_logical(t, _S2_BITS - 1) & ~1
                s2_v[sl] = (t & _S2_MASK) * 2
            lo = pl.ds(0, _SPLIT)
            hi = pl.ds(_SPLIT, _C - _SPLIT)
            cps = [
                pltpu.async_copy(p1_hbm.at[s1_v.at[lo]], buf1.at[lo], sem1),
                pltpu.async_copy(p1_hbm.at[s1_v.at[hi]], buf1.at[hi], sem1),
                pltpu.async_copy(p2_hbm.at[s2_v.at[lo]], buf2.at[lo], sem2),
                pltpu.async_copy(p2_hbm.at[s2_v.at[hi]], buf2.at[hi], sem2),
            ]
            for cp in cps:
                cp.wait()

            def add_body(i, c):
                for k in range(_D // 16):
                    sl = pl.ds(k * 16, 16)
                    buf2[i, sl] = buf2[i, sl] + buf1[i, sl]
                return c

            lax.fori_loop(0, _C, add_body, 0)
            pltpu.sync_copy(buf2, out_hbm.at[b])
            return carry

        lax.fori_loop(0, seq_per_w, chunk_body, 0)

    return sc_gather


def kernel(token_ids, emb_s1, emb_s2, W, b):
    B, L = token_ids.shape
    n = B * L
    n1 = emb_s1.shape[0]
    n2 = emb_s2.shape[0]
    scale = math.sqrt(_D)
    zpad = jnp.zeros((_D, _D), jnp.float32)
    w1 = jnp.concatenate([W[:_D] * scale, zpad], axis=1)
    w2 = jnp.concatenate([W[_D:] * scale, zpad], axis=1)
    bpad = jnp.concatenate([b, jnp.zeros((_D,), jnp.float32)])

    p1 = pl.pallas_call(
        _proj_t_bias_kernel,
        out_shape=jax.ShapeDtypeStruct((n1, 2 * _D), jnp.float32),
    )(emb_s1.T, w1, bpad.reshape(1, 2 * _D))

    blk = 8192
    p2 = pl.pallas_call(
        _proj_t_kernel,
        grid=(n2 // blk,),
        in_specs=[
            pl.BlockSpec((_D, blk), lambda i: (0, i)),
            pl.BlockSpec((_D, 2 * _D), lambda i: (0, 0)),
        ],
        out_specs=pl.BlockSpec((blk, 2 * _D), lambda i: (i, 0)),
        out_shape=jax.ShapeDtypeStruct((n2, 2 * _D), jnp.float32),
    )(emb_s2.T, w2)

    out = _make_sc_gather(B, L)(
        token_ids.reshape(n),
        p1.reshape(n1 * 2, _D),
        p2.reshape(n2 * 2, _D),
    )
    return out
